# Initial kernel scaffold; baseline (speedup 1.0000x reference)
import jax
import jax.numpy as jnp
from jax.experimental import pallas as pl


def kernel(X, embedding, positional_embedding):
    raise NotImplementedError("write your pallas kernel here")


# speedup vs baseline: 2.1375x; 2.1375x over previous
import functools

import jax
import jax.numpy as jnp
from jax import lax
from jax.experimental import pallas as pl
from jax.experimental.pallas import tpu as pltpu
from jax.experimental.pallas import tpu_sc as plsc

N_ITEMS = 100000
D = 128
N = 50
BATCH = 4096

NC = 2
NS = 16
NW = NC * NS
ROWS_PER_W = BATCH * N // NW
CHUNK = 128
NCH = ROWS_PER_W // CHUNK
NBUF = 5
NROUND = NCH // NBUF
POS_TILE = 4 * N
LANES = 16


def _build_kernel():
    mesh = plsc.VectorSubcoreMesh(core_axis_name="c", subcore_axis_name="s")

    @functools.partial(
        pl.kernel,
        mesh=mesh,
        out_type=jax.ShapeDtypeStruct((BATCH * N, D), jnp.float32),
        scratch_types=(
            [pltpu.VMEM((NCH, CHUNK), jnp.int32),
             pltpu.VMEM((POS_TILE, D), jnp.float32),
             pltpu.VMEM((NBUF, CHUNK, D), jnp.float32)]
            + [pltpu.SemaphoreType.DMA] * (2 * NBUF)
        ),
    )
    def emb_kernel(x_hbm, table_hbm, pos_hbm, out_hbm, idx_v, pos4, buf, *sems):
        gsem = sems[:NBUF]
        ssem = sems[NBUF:]
        wid = lax.axis_index("s") * NC + lax.axis_index("c")
        base = wid * ROWS_PER_W

        pltpu.sync_copy(x_hbm.at[wid], idx_v)
        for k in range(POS_TILE // N):
            pltpu.sync_copy(pos_hbm, pos4.at[pl.ds(k * N, N)])

        def g_issue(cg, b):
            pltpu.async_copy(table_hbm.at[idx_v.at[cg]], buf.at[b], gsem[b])

        def g_wait(cg, b):
            pltpu.make_async_copy(table_hbm.at[idx_v.at[cg]], buf.at[b],
                                  gsem[b]).wait()

        def s_issue(cg, b):
            row0 = base + cg * CHUNK
            pltpu.async_copy(buf.at[b], out_hbm.at[pl.ds(row0, CHUNK)],
                             ssem[b])

        def s_wait(b):
            pltpu.make_async_copy(buf.at[b], out_hbm.at[pl.ds(0, CHUNK)],
                                  ssem[b]).wait()

        def add_pos(cg, b):
            if isinstance(cg, int):
                p0 = (cg * CHUNK) % N
            else:
                p0 = lax.rem(cg * CHUNK, N)

            def row(r, carry):
                src = p0 + r
                for c in range(D // LANES):
                    sl = pl.ds(c * LANES, LANES)
                    buf[b, r, sl] = buf[b, r, sl] + pos4[src, sl]
                return carry

            lax.fori_loop(0, CHUNK, row, 0)

        def process(cg, b):
            b2 = (b + 2) % NBUF
            if isinstance(cg, int):
                if cg >= 3:
                    s_wait(b2)
                if cg + 2 < NCH:
                    g_issue(cg + 2, b2)
            else:
                s_wait(b2)
                g_issue(cg + 2, b2)
            g_wait(cg, b)
            add_pos(cg, b)
            s_issue(cg, b)

        g_issue(0, 0)
        g_issue(1, 1)
        for b in range(NBUF):
            process(b, b)

        def round_body(r, carry):
            cg0 = r * NBUF
            for b in range(NBUF):
                process(cg0 + b, b)
            return carry

        lax.fori_loop(1, NROUND - 1, round_body, 0)

        for b in range(NBUF):
            process((NROUND - 1) * NBUF + b, b)
        for b in range(2, NBUF):
            s_wait(b)

    return emb_kernel


_EMB_KERNEL = _build_kernel()


def kernel(X, embedding, positional_embedding):
    Xr = X.astype(jnp.int32).reshape(NW, NCH, CHUNK)
    out = _EMB_KERNEL(Xr, embedding, positional_embedding)
    return out.reshape(BATCH, N, D)


# speedup vs baseline: 5.6045x; 2.6220x over previous
import functools

import jax
import jax.numpy as jnp
from jax import lax
from jax.experimental import pallas as pl
from jax.experimental.pallas import tpu as pltpu
from jax.experimental.pallas import tpu_sc as plsc

N_ITEMS = 100000
D = 128
N = 50
NPAD = 64
BATCH = 4096

NC = 2
NS = 16
NW = NC * NS
B_PER_W = BATCH // NW
SUB = 2
NCH = B_PER_W // SUB
NBUF = 4
NROUND = NCH // NBUF
LANES = 16


def _build_kernel():
    mesh = plsc.VectorSubcoreMesh(core_axis_name="c", subcore_axis_name="s")

    @functools.partial(
        pl.kernel,
        mesh=mesh,
        out_type=jax.ShapeDtypeStruct((BATCH, N, D), jnp.float32),
        scratch_types=(
            [pltpu.VMEM((B_PER_W, NPAD), jnp.int32),
             pltpu.VMEM((N, D), jnp.float32),
             pltpu.VMEM((NBUF, SUB, N, D), jnp.float32)]
            + [pltpu.SemaphoreType.DMA] * (2 * NBUF)
        ),
    )
    def emb_kernel(x_hbm, table_hbm, pos_hbm, out_hbm, idx_v, pos_v, buf,
                   *sems):
        gsem = sems[:NBUF]
        ssem = sems[NBUF:]
        wid = lax.axis_index("s") * NC + lax.axis_index("c")
        bat0 = wid * B_PER_W

        pltpu.sync_copy(x_hbm.at[wid], idx_v)
        pltpu.sync_copy(pos_hbm, pos_v)

        def g_issue(cg, b):
            for s in range(SUB):
                pltpu.async_copy(
                    table_hbm.at[idx_v.at[cg * SUB + s, pl.ds(0, N)]],
                    buf.at[b, s], gsem[b])

        def g_wait(cg, b):
            for s in range(SUB):
                pltpu.make_async_copy(
                    table_hbm.at[idx_v.at[cg * SUB + s, pl.ds(0, N)]],
                    buf.at[b, s], gsem[b]).wait()

        def s_issue(cg, b):
            pltpu.async_copy(buf.at[b], out_hbm.at[pl.ds(bat0 + cg * SUB, SUB)],
                             ssem[b])

        def s_wait(b):
            pltpu.make_async_copy(buf.at[b], out_hbm.at[pl.ds(0, SUB)],
                                  ssem[b]).wait()

        def add_pos(b):
            def row(r, carry):
                for c in range(D // LANES):
                    sl = pl.ds(c * LANES, LANES)
                    p = pos_v[r, sl]
                    for s in range(SUB):
                        buf[b, s, r, sl] = buf[b, s, r, sl] + p
                return carry

            lax.fori_loop(0, N, row, 0)

        def process(cg, b):
            bn = (b + 1) % NBUF
            static = isinstance(cg, int)
            if not static or 3 <= cg:
                s_wait(bn)
            if not static or cg + 1 < NCH:
                g_issue(cg + 1, bn)
            g_wait(cg, b)
            add_pos(b)
            s_issue(cg, b)

        g_issue(0, 0)
        for b in range(NBUF):
            process(b, b)

        def round_body(r, carry):
            cg0 = r * NBUF
            for b in range(NBUF):
                process(cg0 + b, b)
            return carry

        lax.fori_loop(1, NROUND - 1, round_body, 0)

        for b in range(NBUF):
            cg = (NROUND - 1) * NBUF + b
            bn = (b + 1) % NBUF
            s_wait(bn)
            if cg + 1 < NCH:
                g_issue(cg + 1, bn)
            g_wait(cg, b)
            add_pos(b)
            s_issue(cg, b)
        for b in range(1, NBUF):
            s_wait(b)

    return emb_kernel


_EMB_KERNEL = _build_kernel()


def kernel(X, embedding, positional_embedding):
    Xp = jnp.pad(X.astype(jnp.int32), ((0, 0), (0, NPAD - N)))
    Xr = Xp.reshape(NW, B_PER_W, NPAD)
    return _EMB_KERNEL(Xr, embedding, positional_embedding)


# speedup vs baseline: 5.6145x; 1.0018x over previous
import functools

import jax
import jax.numpy as jnp
from jax import lax
from jax.experimental import pallas as pl
from jax.experimental.pallas import tpu as pltpu
from jax.experimental.pallas import tpu_sc as plsc

N_ITEMS = 100000
D = 128
N = 50
NPAD = 64
BATCH = 4096

NC = 2
NS = 16
NW = NC * NS
B_PER_W = BATCH // NW
SUB = 2
NCH = B_PER_W // SUB
NBUF = 4
NROUND = NCH // NBUF
LANES = 16


def _build_kernel():
    mesh = plsc.VectorSubcoreMesh(core_axis_name="c", subcore_axis_name="s")

    @functools.partial(
        pl.kernel,
        mesh=mesh,
        compiler_params=pltpu.CompilerParams(use_tc_tiling_on_sc=True),
        out_type=jax.ShapeDtypeStruct((BATCH, N, D), jnp.float32),
        scratch_types=(
            [pltpu.VMEM((B_PER_W * NPAD // D, D), jnp.int32),
             pltpu.VMEM((N, D), jnp.float32),
             pltpu.VMEM((NBUF, SUB, N, D), jnp.float32)]
            + [pltpu.SemaphoreType.DMA] * (2 * NBUF)
        ),
    )
    def emb_kernel(x_hbm, table_hbm, pos_hbm, out_hbm, idx_v, pos_v, buf,
                   *sems):
        gsem = sems[:NBUF]
        ssem = sems[NBUF:]
        wid = lax.axis_index("s") * NC + lax.axis_index("c")
        bat0 = wid * B_PER_W

        pltpu.sync_copy(x_hbm.at[wid], idx_v)
        pltpu.sync_copy(pos_hbm, pos_v)

        def g_issue(cg, b):
            for s in range(SUB):
                pltpu.async_copy(
                    table_hbm.at[idx_v.at[cg, pl.ds(s * NPAD, N)]],
                    buf.at[b, s], gsem[b])

        def g_wait(cg, b):
            for s in range(SUB):
                pltpu.make_async_copy(
                    table_hbm.at[idx_v.at[cg, pl.ds(s * NPAD, N)]],
                    buf.at[b, s], gsem[b]).wait()

        def s_issue(cg, b):
            pltpu.async_copy(buf.at[b], out_hbm.at[pl.ds(bat0 + cg * SUB, SUB)],
                             ssem[b])

        def s_wait(b):
            pltpu.make_async_copy(buf.at[b], out_hbm.at[pl.ds(0, SUB)],
                                  ssem[b]).wait()

        def add_pos(b):
            def row(r, carry):
                for c in range(D // LANES):
                    sl = pl.ds(c * LANES, LANES)
                    p = pos_v[r, sl]
                    for s in range(SUB):
                        buf[b, s, r, sl] = buf[b, s, r, sl] + p
                return carry

            lax.fori_loop(0, N, row, 0)

        def process(cg, b):
            bn = (b + 1) % NBUF
            static = isinstance(cg, int)
            if not static or 3 <= cg:
                s_wait(bn)
            if not static or cg + 1 < NCH:
                g_issue(cg + 1, bn)
            g_wait(cg, b)
            add_pos(b)
            s_issue(cg, b)

        g_issue(0, 0)
        for b in range(NBUF):
            process(b, b)

        def round_body(r, carry):
            cg0 = r * NBUF
            for b in range(NBUF):
                process(cg0 + b, b)
            return carry

        lax.fori_loop(1, NROUND - 1, round_body, 0)

        for b in range(NBUF):
            cg = (NROUND - 1) * NBUF + b
            bn = (b + 1) % NBUF
            s_wait(bn)
            if cg + 1 < NCH:
                g_issue(cg + 1, bn)
            g_wait(cg, b)
            add_pos(b)
            s_issue(cg, b)
        for b in range(1, NBUF):
            s_wait(b)

    return emb_kernel


_EMB_KERNEL = _build_kernel()


def kernel(X, embedding, positional_embedding):
    Xp = jnp.pad(X.astype(jnp.int32), ((0, 0), (0, NPAD - N)))
    Xr = Xp.reshape(NW, B_PER_W * NPAD // D, D)
    return _EMB_KERNEL(Xr, embedding, positional_embedding)


# speedup vs baseline: 10.0350x; 1.7873x over previous
import functools

import jax
import jax.numpy as jnp
from jax import lax
from jax.experimental import pallas as pl
from jax.experimental.pallas import tpu as pltpu
from jax.experimental.pallas import tpu_sc as plsc

N_ITEMS = 100000
D = 128
N = 50
BATCH = 4096

NC = 2
NS = 16
NW = NC * NS
ROWS = BATCH * N
ROWS_PER_W = ROWS // NW
CHUNK = 128
NCH = ROWS_PER_W // CHUNK
PLANES_PER_CHUNK = BATCH // CHUNK
NBUF = 5
NROUND = NCH // NBUF
LANES = 16


def _build_kernel():
    mesh = plsc.VectorSubcoreMesh(core_axis_name="c", subcore_axis_name="s")

    @functools.partial(
        pl.kernel,
        mesh=mesh,
        out_type=jax.ShapeDtypeStruct((ROWS, D), jnp.float32),
        scratch_types=(
            [pltpu.VMEM((ROWS_PER_W,), jnp.int32),
             pltpu.VMEM((N, D), jnp.float32),
             pltpu.VMEM((NBUF, CHUNK, D), jnp.float32)]
            + [pltpu.SemaphoreType.DMA] * (2 * NBUF)
        ),
    )
    def emb_kernel(x_hbm, table_hbm, pos_hbm, out_hbm, idx_v, pos_v, buf,
                   *sems):
        gsem = sems[:NBUF]
        ssem = sems[NBUF:]
        wid = lax.axis_index("s") * NC + lax.axis_index("c")
        row0 = wid * ROWS_PER_W

        pltpu.sync_copy(x_hbm.at[wid], idx_v)
        pltpu.sync_copy(pos_hbm, pos_v)

        def g_issue(cg, b):
            pltpu.async_copy(table_hbm.at[idx_v.at[pl.ds(cg * CHUNK, CHUNK)]],
                             buf.at[b], gsem[b])

        def g_wait(cg, b):
            pltpu.make_async_copy(
                table_hbm.at[idx_v.at[pl.ds(cg * CHUNK, CHUNK)]],
                buf.at[b], gsem[b]).wait()

        def s_issue(cg, b):
            pltpu.async_copy(buf.at[b],
                             out_hbm.at[pl.ds(row0 + cg * CHUNK, CHUNK)],
                             ssem[b])

        def s_wait(b):
            pltpu.make_async_copy(buf.at[b], out_hbm.at[pl.ds(0, CHUNK)],
                                  ssem[b]).wait()

        def add_pos(cg, b):
            n_row = (wid * NCH + cg) // PLANES_PER_CHUNK
            ps = [pos_v[n_row, pl.ds(c * LANES, LANES)]
                  for c in range(D // LANES)]

            def row(r, carry):
                for c in range(D // LANES):
                    sl = pl.ds(c * LANES, LANES)
                    buf[b, r, sl] = buf[b, r, sl] + ps[c]
                return carry

            lax.fori_loop(0, CHUNK, row, 0)

        def process(cg, b):
            b2 = (b + 2) % NBUF
            static = isinstance(cg, int)
            if not static or cg >= 3:
                s_wait(b2)
            if not static or cg + 2 < NCH:
                g_issue(cg + 2, b2)
            g_wait(cg, b)
            add_pos(cg, b)
            s_issue(cg, b)

        g_issue(0, 0)
        g_issue(1, 1)
        for b in range(NBUF):
            process(b, b)

        def round_body(r, carry):
            cg0 = r * NBUF
            for b in range(NBUF):
                process(cg0 + b, b)
            return carry

        lax.fori_loop(1, NROUND - 1, round_body, 0)

        for b in range(NBUF):
            process((NROUND - 1) * NBUF + b, b)
        for b in range(2, NBUF):
            s_wait(b)

    return emb_kernel


_EMB_KERNEL = _build_kernel()


def kernel(X, embedding, positional_embedding):
    Xt = jnp.transpose(X.astype(jnp.int32)).reshape(NW, ROWS_PER_W)
    out = _EMB_KERNEL(Xt, embedding, positional_embedding)
    return jnp.transpose(out.reshape(N, BATCH, D), (1, 0, 2))


# speedup vs baseline: 10.0593x; 1.0024x over previous
import functools

import jax
import jax.numpy as jnp
from jax import lax
from jax.experimental import pallas as pl
from jax.experimental.pallas import tpu as pltpu
from jax.experimental.pallas import tpu_sc as plsc

N_ITEMS = 100000
D = 128
N = 50
NPOS_PAD = 64
BATCH = 4096

NC = 2
NS = 16
NW = NC * NS
ROWS = BATCH * N
ROWS_PER_W = ROWS // NW
CHUNK = 256
GSTREAM = 128
NCH = ROWS_PER_W // CHUNK
NBUF = 3
LANES = 16


def _build_kernel():
    mesh = plsc.VectorSubcoreMesh(core_axis_name="c", subcore_axis_name="s")

    @functools.partial(
        pl.kernel,
        mesh=mesh,
        out_type=jax.ShapeDtypeStruct((ROWS, D), jnp.float32),
        scratch_types=(
            [pltpu.VMEM((ROWS_PER_W,), jnp.int32),
             pltpu.VMEM((16, D), jnp.float32),
             pltpu.VMEM((NBUF, CHUNK, D), jnp.float32)]
            + [pltpu.SemaphoreType.DMA] * (2 * NBUF)
        ),
    )
    def emb_kernel(x_hbm, table_hbm, pos_hbm, out_hbm, idx_v, pos_v, buf,
                   *sems):
        gsem = sems[:NBUF]
        ssem = sems[NBUF:]
        wid = lax.axis_index("s") * NC + lax.axis_index("c")
        row0 = wid * ROWS_PER_W
        n0 = pl.multiple_of(((NCH * wid) // (BATCH // CHUNK)) & ~7, 8)

        pltpu.sync_copy(x_hbm.at[wid], idx_v)
        pltpu.sync_copy(pos_hbm.at[pl.ds(n0, 16)], pos_v)

        def g_issue(cg, b):
            for s in range(CHUNK // GSTREAM):
                pltpu.async_copy(
                    table_hbm.at[idx_v.at[pl.ds(cg * CHUNK + s * GSTREAM,
                                                GSTREAM)]],
                    buf.at[b, pl.ds(s * GSTREAM, GSTREAM)], gsem[b])

        def g_wait(cg, b):
            for s in range(CHUNK // GSTREAM):
                pltpu.make_async_copy(
                    table_hbm.at[idx_v.at[pl.ds(cg * CHUNK + s * GSTREAM,
                                                GSTREAM)]],
                    buf.at[b, pl.ds(s * GSTREAM, GSTREAM)], gsem[b]).wait()

        def s_issue(cg, b):
            pltpu.async_copy(buf.at[b],
                             out_hbm.at[pl.ds(row0 + cg * CHUNK, CHUNK)],
                             ssem[b])

        def s_wait(b):
            pltpu.make_async_copy(buf.at[b], out_hbm.at[pl.ds(0, CHUNK)],
                                  ssem[b]).wait()

        def add_pos(cg, b):
            n_loc = (NCH * wid + cg) // (BATCH // CHUNK) - n0
            ps = [pos_v[n_loc, pl.ds(c * LANES, LANES)]
                  for c in range(D // LANES)]

            def row(r, carry):
                for c in range(D // LANES):
                    sl = pl.ds(c * LANES, LANES)
                    buf[b, r, sl] = buf[b, r, sl] + ps[c]
                return carry

            lax.fori_loop(0, CHUNK, row, 0)

        def process(cg, b):
            bn = (b + 1) % NBUF
            static = isinstance(cg, int)
            if not static or cg >= 2:
                s_wait(bn)
            if not static or cg + 1 < NCH:
                g_issue(cg + 1, bn)
            g_wait(cg, b)
            add_pos(cg, b)
            s_issue(cg, b)

        g_issue(0, 0)
        for b in range(NBUF):
            process(b, b)

        def round_body(r, carry):
            cg0 = r * NBUF
            for b in range(NBUF):
                process(cg0 + b, b)
            return carry

        lax.fori_loop(1, (NCH - 1) // NBUF, round_body, 0)

        process(NCH - 1, (NCH - 1) % NBUF)
        s_wait((NCH - 2) % NBUF)
        s_wait((NCH - 1) % NBUF)

    return emb_kernel


_EMB_KERNEL = _build_kernel()


def kernel(X, embedding, positional_embedding):
    Xt = jnp.transpose(X.astype(jnp.int32)).reshape(NW, ROWS_PER_W)
    pos_pad = jnp.pad(positional_embedding, ((0, NPOS_PAD - N), (0, 0)))
    out = _EMB_KERNEL(Xt, embedding, pos_pad)
    return jnp.transpose(out.reshape(N, BATCH, D), (1, 0, 2))
